# baseline (speedup 1.0000x reference)
import jax
import jax.numpy as jnp
from jax import lax
from jax.experimental import pallas as pl
from jax.experimental.pallas import tpu as pltpu

_EMB = 768
_T = 2048
_KK = 32
_BLK = 256
_NB = _T // _BLK
_MPB = _BLK // _KK
_NM = _T // _KK
_BF = jnp.bfloat16


def _dot(a, b):
    return jnp.dot(a, b, preferred_element_type=jnp.float32)


def _attn_kernel(x_ref, WkT_ref, WqT_ref, WvT_ref, Wu1T_ref, Wu2T_ref,
                 bk_ref, bq_ref, bv_ref, bu_ref, out_ref, qs_ref, vs_ref):
    i = pl.program_id(1)

    @pl.when(i == 0)
    def _():
        qs_ref[...] = jnp.zeros((_NM, _EMB), jnp.float32)
        vs_ref[...] = jnp.zeros((_NM, _EMB), jnp.float32)

    xb = x_ref[0]
    K = _dot(xb, WkT_ref[...]) + bk_ref[...]
    Q = _dot(xb, WqT_ref[...]) + bq_ref[...]
    V = _dot(xb, WvT_ref[...]) + bv_ref[...]
    Kb = K.astype(_BF)
    Qb = Q.astype(_BF)
    Vb = V.astype(_BF)

    qs_ref[pl.ds(i * _MPB, _MPB), :] = Q.reshape(_MPB, _KK, _EMB)[:, 0, :]
    vs_ref[pl.ds(i * _MPB, _MPB), :] = V.reshape(_MPB, _KK, _EMB)[:, 0, :]

    S = _dot(Kb, Qb.T)
    r = lax.broadcasted_iota(jnp.int32, (_BLK, _BLK), 0)
    c = lax.broadcasted_iota(jnp.int32, (_BLK, _BLK), 1)
    mask1 = (r // _KK == c // _KK) & (c <= r)
    h1 = _dot(jnp.where(mask1, S, 0.0).astype(_BF), Vb)

    D = _dot(Kb, qs_ref[...].astype(_BF).T)
    rj = lax.broadcasted_iota(jnp.int32, (_BLK, _NM), 0)
    cm = lax.broadcasted_iota(jnp.int32, (_BLK, _NM), 1)
    mask2 = (cm * _KK) <= (i * _BLK + rj)
    h2 = _dot(jnp.where(mask2, D, 0.0).astype(_BF), vs_ref[...].astype(_BF))

    out_ref[0] = (_dot(h1.astype(_BF), Wu1T_ref[...])
                  + _dot(h2.astype(_BF), Wu2T_ref[...])
                  + bu_ref[...])


def kernel(x, Wk, bk, Wq, bq, Wv, bv, Wu, bu):
    B = x.shape[0]
    wspec = pl.BlockSpec((_EMB, _EMB), lambda b, i: (0, 0))
    bspec = pl.BlockSpec((1, _EMB), lambda b, i: (0, 0))
    return pl.pallas_call(
        _attn_kernel,
        grid=(B, _NB),
        in_specs=[
            pl.BlockSpec((1, _BLK, _EMB), lambda b, i: (b, i, 0)),
            wspec, wspec, wspec, wspec, wspec,
            bspec, bspec, bspec, bspec,
        ],
        out_specs=pl.BlockSpec((1, _BLK, _EMB), lambda b, i: (b, i, 0)),
        out_shape=jax.ShapeDtypeStruct((B, _T, _EMB), jnp.float32),
        scratch_shapes=[
            pltpu.VMEM((_NM, _EMB), jnp.float32),
            pltpu.VMEM((_NM, _EMB), jnp.float32),
        ],
        compiler_params=pltpu.CompilerParams(
            dimension_semantics=("parallel", "arbitrary")),
    )(x.astype(_BF), Wk.T.astype(_BF), Wq.T.astype(_BF), Wv.T.astype(_BF),
      Wu[:, :_EMB].T.astype(_BF), Wu[:, _EMB:].T.astype(_BF),
      bk.reshape(1, _EMB), bq.reshape(1, _EMB), bv.reshape(1, _EMB),
      bu.reshape(1, _EMB))


# speedup vs baseline: 1.1806x; 1.1806x over previous
import jax
import jax.numpy as jnp
from jax import lax
from jax.experimental import pallas as pl
from jax.experimental.pallas import tpu as pltpu

_EMB = 768
_T = 2048
_KK = 32
_BLK = 256
_NB = _T // _BLK
_MPB = _BLK // _KK
_NM = _T // _KK
_BF = jnp.bfloat16

_DN_T = (((1,), (1,)), ((), ()))


def _dot_t(a, b):
    return lax.dot_general(a, b, _DN_T, preferred_element_type=jnp.float32)


def _dot(a, b):
    return jnp.dot(a, b, preferred_element_type=jnp.float32)


def _attn_kernel(x_ref, Wk_ref, Wq_ref, Wv_ref, Wu_ref,
                 bk_ref, bq_ref, bv_ref, bu_ref, out_ref,
                 wk_ref, wq_ref, wv_ref, wu1_ref, wu2_ref, qs_ref, vs_ref):
    b = pl.program_id(0)
    i = pl.program_id(1)

    @pl.when((b == 0) & (i == 0))
    def _():
        wk_ref[...] = Wk_ref[...].astype(_BF)
        wq_ref[...] = Wq_ref[...].astype(_BF)
        wv_ref[...] = Wv_ref[...].astype(_BF)
        wu1_ref[...] = Wu_ref[:, :_EMB].astype(_BF)
        wu2_ref[...] = Wu_ref[:, _EMB:].astype(_BF)

    @pl.when(i == 0)
    def _():
        qs_ref[...] = jnp.zeros((_NM, _EMB), jnp.float32)
        vs_ref[...] = jnp.zeros((_NM, _EMB), jnp.float32)

    xb = x_ref[0].astype(_BF)
    K = _dot_t(xb, wk_ref[...]) + bk_ref[...]
    Q = _dot_t(xb, wq_ref[...]) + bq_ref[...]
    V = _dot_t(xb, wv_ref[...]) + bv_ref[...]
    Kb = K.astype(_BF)
    Qb = Q.astype(_BF)
    Vb = V.astype(_BF)

    qs_ref[pl.ds(i * _MPB, _MPB), :] = Q.reshape(_MPB, _KK, _EMB)[:, 0, :]
    vs_ref[pl.ds(i * _MPB, _MPB), :] = V.reshape(_MPB, _KK, _EMB)[:, 0, :]

    S = _dot_t(Kb, Qb)
    r = lax.broadcasted_iota(jnp.int32, (_BLK, _BLK), 0)
    c = lax.broadcasted_iota(jnp.int32, (_BLK, _BLK), 1)
    mask1 = (r // _KK == c // _KK) & (c <= r)
    h1 = _dot(jnp.where(mask1, S, 0.0).astype(_BF), Vb)

    D = _dot_t(Kb, qs_ref[...].astype(_BF))
    rj = lax.broadcasted_iota(jnp.int32, (_BLK, _NM), 0)
    cm = lax.broadcasted_iota(jnp.int32, (_BLK, _NM), 1)
    mask2 = (cm * _KK) <= (i * _BLK + rj)
    h2 = _dot(jnp.where(mask2, D, 0.0).astype(_BF), vs_ref[...].astype(_BF))

    out_ref[0] = (_dot_t(h1.astype(_BF), wu1_ref[...])
                  + _dot_t(h2.astype(_BF), wu2_ref[...])
                  + bu_ref[...])


def kernel(x, Wk, bk, Wq, bq, Wv, bv, Wu, bu):
    B = x.shape[0]
    wspec = pl.BlockSpec((_EMB, _EMB), lambda b, i: (0, 0))
    bspec = pl.BlockSpec((1, _EMB), lambda b, i: (0, 0))
    wscratch = pltpu.VMEM((_EMB, _EMB), _BF)
    return pl.pallas_call(
        _attn_kernel,
        grid=(B, _NB),
        in_specs=[
            pl.BlockSpec((1, _BLK, _EMB), lambda b, i: (b, i, 0)),
            wspec, wspec, wspec,
            pl.BlockSpec((_EMB, 2 * _EMB), lambda b, i: (0, 0)),
            bspec, bspec, bspec, bspec,
        ],
        out_specs=pl.BlockSpec((1, _BLK, _EMB), lambda b, i: (b, i, 0)),
        out_shape=jax.ShapeDtypeStruct((B, _T, _EMB), jnp.float32),
        scratch_shapes=[
            wscratch, wscratch, wscratch, wscratch, wscratch,
            pltpu.VMEM((_NM, _EMB), jnp.float32),
            pltpu.VMEM((_NM, _EMB), jnp.float32),
        ],
    )(x, Wk, Wq, Wv, Wu,
      bk.reshape(1, _EMB), bq.reshape(1, _EMB), bv.reshape(1, _EMB),
      bu.reshape(1, _EMB))
